# Initial kernel scaffold; baseline (speedup 1.0000x reference)
import jax
import jax.numpy as jnp
from jax.experimental import pallas as pl


def kernel(features, blocks, cluster_centers, W1, b1, W2, b2, epoch, max_epochs):
    raise NotImplementedError("write your pallas kernel here")


# speedup vs baseline: 1.0364x; 1.0364x over previous
import jax
import jax.numpy as jnp
from jax.experimental import pallas as pl


def _copy_kernel(blocks_ref, out_ref):
    out_ref[...] = blocks_ref[...]


def kernel(features, blocks, cluster_centers, W1, b1, W2, b2, epoch, max_epochs):
    N, L = blocks.shape
    return pl.pallas_call(
        _copy_kernel,
        out_shape=jax.ShapeDtypeStruct((N, L), blocks.dtype),
    )(blocks)
